# Initial kernel scaffold; baseline (speedup 1.0000x reference)
import jax
import jax.numpy as jnp
from jax import lax
from jax.experimental import pallas as pl
from jax.experimental.pallas import tpu as pltpu
from jax.experimental.pallas import tpu_sc as plsc

NC = 2
NS = 16
NW = NC * NS
CHUNK = 128


def _body(x_hbm, table_hbm, out_hbm, idx_v, rows0, rows1,
          gsem0, gsem1, wsem0, wsem1):
    wid = lax.axis_index("s") * NC + lax.axis_index("c")
    nchunks = x_hbm.shape[1]

    pltpu.sync_copy(x_hbm.at[wid], idx_v)

    def step(i, carry):
        c0 = 2 * i
        c1 = c0 + 1
        g0 = pltpu.make_async_copy(table_hbm.at[idx_v.at[c0]], rows0, gsem0)
        g1 = pltpu.make_async_copy(table_hbm.at[idx_v.at[c1]], rows1, gsem1)
        g0.start()
        g1.start()
        g0.wait()
        w0 = pltpu.make_async_copy(rows0, out_hbm.at[wid, c0], wsem0)
        w0.start()
        g1.wait()
        w1 = pltpu.make_async_copy(rows1, out_hbm.at[wid, c1], wsem1)
        w1.start()
        w0.wait()
        w1.wait()
        return carry

    lax.fori_loop(0, nchunks // 2, step, 0)


def kernel(x, table):
    B0, B1 = x.shape
    V, D = table.shape
    total = B0 * B1
    nchunks = total // (NW * CHUNK)
    x3 = x.reshape(NW, nchunks, CHUNK)

    fn = pl.kernel(
        _body,
        out_type=jax.ShapeDtypeStruct((NW, nchunks, CHUNK, D), jnp.float32),
        mesh=plsc.VectorSubcoreMesh(core_axis_name="c", subcore_axis_name="s"),
        scratch_types=[
            pltpu.VMEM((nchunks, CHUNK), jnp.int32),
            pltpu.VMEM((CHUNK, D), jnp.float32),
            pltpu.VMEM((CHUNK, D), jnp.float32),
            pltpu.SemaphoreType.DMA,
            pltpu.SemaphoreType.DMA,
            pltpu.SemaphoreType.DMA,
            pltpu.SemaphoreType.DMA,
        ],
    )
    out4 = fn(x3, table)
    return out4.reshape(B0, B1, D)


# speedup vs baseline: 1.7807x; 1.7807x over previous
import jax
import jax.numpy as jnp
from jax import lax
from jax.experimental import pallas as pl
from jax.experimental.pallas import tpu as pltpu
from jax.experimental.pallas import tpu_sc as plsc

NC = 2
NS = 16
NW = NC * NS
CHUNK = 128


def _body(x_hbm, table_hbm, out_hbm, idx_v, rows0, rows1,
          gsem0, gsem1, wsem0, wsem1):
    wid = lax.axis_index("s") * NC + lax.axis_index("c")
    nchunks = x_hbm.shape[1]

    pltpu.sync_copy(x_hbm.at[wid], idx_v)

    def step(i, carry):
        c0 = 2 * i
        c1 = c0 + 1
        g0 = pltpu.make_async_copy(table_hbm.at[idx_v.at[c0]], rows0, gsem0)
        g1 = pltpu.make_async_copy(table_hbm.at[idx_v.at[c1]], rows1, gsem1)
        g0.start()
        g1.start()
        g0.wait()
        w0 = pltpu.make_async_copy(rows0, out_hbm.at[wid, c0], wsem0)
        w0.start()
        g1.wait()
        w1 = pltpu.make_async_copy(rows1, out_hbm.at[wid, c1], wsem1)
        w1.start()
        w0.wait()
        w1.wait()
        return carry

    lax.fori_loop(0, nchunks // 2, step, 0)


def kernel(x, table):
    B0, B1 = x.shape
    V, D = table.shape
    total = B0 * B1
    nchunks = total // (NW * CHUNK)
    x3 = x.reshape(NW, nchunks, CHUNK)

    fn = pl.kernel(
        _body,
        out_type=jax.ShapeDtypeStruct((NW, nchunks, CHUNK, D), jnp.float32),
        mesh=plsc.VectorSubcoreMesh(core_axis_name="c", subcore_axis_name="s"),
        compiler_params=pltpu.CompilerParams(use_tc_tiling_on_sc=False),
        scratch_types=[
            pltpu.VMEM((nchunks, CHUNK), jnp.int32),
            pltpu.VMEM((CHUNK, D), jnp.float32),
            pltpu.VMEM((CHUNK, D), jnp.float32),
            pltpu.SemaphoreType.DMA,
            pltpu.SemaphoreType.DMA,
            pltpu.SemaphoreType.DMA,
            pltpu.SemaphoreType.DMA,
        ],
    )
    out4 = fn(x3, table)
    return out4.reshape(B0, B1, D)
